# baseline (speedup 1.0000x reference)
import dataclasses
import functools

import jax
import jax.numpy as jnp
from jax import lax
from jax.experimental import pallas as pl
from jax.experimental.pallas import tpu as pltpu
from jax.experimental.pallas import tpu_sc as plsc

N_NODES = 10000
NODE_DIM = 128
EDGE_DIM = 4
N_EDGES = 320000
TWO_D = 2 * NODE_DIM

NC = 2
NS = 16
NW = NC * NS
EPT = N_EDGES // NW
EB = 40
NBLK = EPT // EB
ACC_ROWS = 10240
ROWS_PER_TILE = ACC_ROWS // NS
ZROWS = 16


def _tc_precompute(x, WsT, WrT, brow):
    blk = 1000

    def body(x_ref, ws_ref, wr_ref, b_ref, xs_ref, xr_ref):
        xb = x_ref[...]
        xs_ref[...] = jnp.dot(xb, ws_ref[...], preferred_element_type=jnp.float32)
        xr_ref[...] = (jnp.dot(xb, wr_ref[...], preferred_element_type=jnp.float32)
                       + b_ref[...])

    return pl.pallas_call(
        body,
        grid=(N_NODES // blk,),
        in_specs=[
            pl.BlockSpec((blk, NODE_DIM), lambda i: (i, 0)),
            pl.BlockSpec((NODE_DIM, TWO_D), lambda i: (0, 0)),
            pl.BlockSpec((NODE_DIM, TWO_D), lambda i: (0, 0)),
            pl.BlockSpec((1, TWO_D), lambda i: (0, 0)),
        ],
        out_specs=[
            pl.BlockSpec((blk, TWO_D), lambda i: (i, 0)),
            pl.BlockSpec((blk, TWO_D), lambda i: (i, 0)),
        ],
        out_shape=[jax.ShapeDtypeStruct((N_NODES, TWO_D), jnp.float32)] * 2,
    )(x, WsT, WrT, brow)


def _tc_combine(partials):
    blk = 1000

    def body(p_ref, o_ref):
        o_ref[...] = p_ref[0] + p_ref[1]

    return pl.pallas_call(
        body,
        grid=(N_NODES // blk,),
        in_specs=[pl.BlockSpec((2, blk, NODE_DIM), lambda i: (0, i, 0))],
        out_specs=pl.BlockSpec((blk, NODE_DIM), lambda i: (i, 0)),
        out_shape=jax.ShapeDtypeStruct((N_NODES, NODE_DIM), jnp.float32),
    )(partials)


def _msg_pair(av, am):
    ax = jnp.abs(av)
    t = jnp.exp(-ax)
    s_ = t / (2.0 + t)
    s2 = s_ * s_
    p = s_ * (2.0 + s2 * (2.0 / 3.0 + s2 * (0.4 + s2 * (2.0 / 7.0))))
    sp = jnp.maximum(av, 0.0) + p
    sg = 1.0 / (1.0 + jnp.exp(-am))
    return sp * sg


def _sc_edges(XS, XR, sidx, ridx, edge_ft, WeT):
    mesh = plsc.VectorSubcoreMesh(core_axis_name="c", subcore_axis_name="s")
    cp = pltpu.CompilerParams()
    if "needs_layout_passes" in pltpu.CompilerParams.__dataclass_fields__:
        cp = dataclasses.replace(cp, needs_layout_passes=False)

    @functools.partial(
        pl.kernel,
        mesh=mesh,
        compiler_params=cp,
        out_type=jax.ShapeDtypeStruct((NC, N_NODES, NODE_DIM), jnp.float32),
        scratch_types=[
            pltpu.VMEM((EB,), jnp.int32),
            pltpu.VMEM((EB,), jnp.int32),
            pltpu.VMEM((EB, TWO_D), jnp.float32),
            pltpu.VMEM((EB, TWO_D), jnp.float32),
            pltpu.VMEM((EB * EDGE_DIM,), jnp.float32),
            pltpu.VMEM((EDGE_DIM, TWO_D), jnp.float32),
            pltpu.VMEM((EB, NODE_DIM), jnp.float32),
            pltpu.VMEM((ZROWS, NODE_DIM), jnp.float32),
            pltpu.VMEM_SHARED((ACC_ROWS, NODE_DIM), jnp.float32),
            pltpu.SemaphoreType.DMA,
        ],
    )
    def k(xs_hbm, xr_hbm, s_hbm, r_hbm, ef_hbm, wet_hbm, out_hbm,
          sb, rb, xsb, xrb, efb, wetb, msgb, zb, acc, sem):
        cid = lax.axis_index("c")
        sid = lax.axis_index("s")
        wid = sid * NC + cid
        base = wid * EPT

        pltpu.sync_copy(wet_hbm, wetb)

        @pl.loop(0, ZROWS)
        def _z(i):
            for j in range(NODE_DIM // 16):
                zb[i, pl.ds(16 * j, 16)] = jnp.zeros((16,), jnp.float32)

        @pl.loop(0, ROWS_PER_TILE // ZROWS)
        def _zc(i):
            pltpu.sync_copy(zb, acc.at[pl.ds(sid * ROWS_PER_TILE + i * ZROWS, ZROWS)])

        plsc.subcore_barrier()

        @pl.loop(0, NBLK)
        def _blk(b):
            off = base + b * EB
            pltpu.sync_copy(s_hbm.at[pl.ds(off, EB)], sb)
            pltpu.sync_copy(r_hbm.at[pl.ds(off, EB)], rb)
            pltpu.async_copy(xs_hbm.at[sb], xsb, sem).wait()
            pltpu.async_copy(xr_hbm.at[rb], xrb, sem).wait()
            pltpu.sync_copy(ef_hbm.at[pl.ds(off * EDGE_DIM, EB * EDGE_DIM)], efb)

            @pl.loop(0, EB)
            def _edge(e):
                ebase = e * EDGE_DIM
                ef = [plsc.load_gather(efb, [jnp.full((16,), ebase + kk, jnp.int32)])
                      for kk in range(EDGE_DIM)]
                for j in range(NODE_DIM // 16):
                    cv = pl.ds(16 * j, 16)
                    cm = pl.ds(NODE_DIM + 16 * j, 16)
                    av = xsb[e, cv] + xrb[e, cv]
                    am = xsb[e, cm] + xrb[e, cm]
                    for kk in range(EDGE_DIM):
                        av = av + ef[kk] * wetb[kk, cv]
                        am = am + ef[kk] * wetb[kk, cm]
                    msgb[e, cv] = _msg_pair(av, am)

            pltpu.sync_copy(msgb, acc.at[rb], add=True)

        plsc.subcore_barrier()

        nchunks = jnp.where(sid == NS - 1, (N_NODES - (NS - 1) * ROWS_PER_TILE) // ZROWS,
                            ROWS_PER_TILE // ZROWS)

        @pl.loop(0, nchunks)
        def _out(i):
            rows = pl.ds(sid * ROWS_PER_TILE + i * ZROWS, ZROWS)
            pltpu.sync_copy(acc.at[rows], out_hbm.at[cid, rows])

    return k(XS, XR, sidx, ridx, edge_ft, WeT)


def kernel(x, edge_index, edge_ft, Wv, bv, Wm, bm):
    s = edge_index[0].astype(jnp.int32)
    r = edge_index[1].astype(jnp.int32)
    WsT = jnp.concatenate([Wv[:, :NODE_DIM].T, Wm[:, :NODE_DIM].T], axis=1)
    WrT = jnp.concatenate([Wv[:, NODE_DIM:TWO_D].T, Wm[:, NODE_DIM:TWO_D].T], axis=1)
    WeT = jnp.concatenate([Wv[:, TWO_D:].T, Wm[:, TWO_D:].T], axis=1)
    brow = jnp.concatenate([bv, bm])[None, :]
    XS, XR = _tc_precompute(x, WsT, WrT, brow)
    partials = _sc_edges(XS, XR, s, r, edge_ft.reshape(-1), WeT)
    return _tc_combine(partials)
